# baseline (speedup 1.0000x reference)
import jax
import jax.numpy as jnp
from jax.experimental import pallas as pl
from jax.experimental.pallas import tpu as pltpu

_IN = 4096
_OUT = 16384
_K = 328
_BN = 512
_BK = 512
_NB = _OUT // _BN
_NK = _IN // _BK
_NTILE = 3328
_NPATCH = 5


def _matvec_body(x_ref, p_ref, m_ref, out_ref):
    k = pl.program_id(1)
    conn = jnp.where(p_ref[...] > 0.5, m_ref[...], 0.0)
    part = jnp.dot(x_ref[...], conn, preferred_element_type=jnp.float32)

    @pl.when(k == 0)
    def _():
        out_ref[...] = part

    @pl.when(k > 0)
    def _():
        out_ref[...] = out_ref[...] + part


def _patch_body(x_ref, p_ref, m_ref, out_ref):
    conn = jnp.where(p_ref[...] > 0.5, m_ref[...], 0.0)
    out_ref[...] = jnp.dot(x_ref[...], conn, preferred_element_type=jnp.float32)


def _select_body(ov_ref, patch_ref, out_ref):
    v = ov_ref[...]
    pieces = []
    for q in range(_NPATCH):
        off = (q % 2) * 256
        pieces.append(patch_ref[0:1, q * _BN + off:q * _BN + off + 256])
        pieces.append(v[:, q * _NTILE + 256:min((q + 1) * _NTILE, _OUT)])
    v = jnp.concatenate(pieces, axis=1)

    bits = jax.lax.bitcast_convert_type(v, jnp.int32)
    k = jnp.int32(_K)

    def val_step(_, carry):
        lo, hi = carry
        mid = lo + (hi - lo) // 2
        cnt = jnp.sum((bits >= mid).astype(jnp.int32))
        ge = cnt >= k
        return jnp.where(ge, mid, lo), jnp.where(ge, hi, mid)

    lo0 = jnp.int32(0)
    hi0 = jnp.int32(0x7F800000)
    t, _ = jax.lax.fori_loop(0, 31, val_step, (lo0, hi0))

    c_gt = jnp.sum((bits > t).astype(jnp.int32))
    need = k - c_gt

    idx = jax.lax.broadcasted_iota(jnp.int32, (1, _OUT), 1)
    tie = bits == t

    def idx_step(_, carry):
        lo, hi = carry
        mid = lo + (hi - lo) // 2
        cnt = jnp.sum((tie & (idx <= mid)).astype(jnp.int32))
        ge = cnt >= need
        return jnp.where(ge, lo, mid), jnp.where(ge, mid, hi)

    _, J = jax.lax.fori_loop(0, 14, idx_step, (jnp.int32(-1), jnp.int32(_OUT - 1)))

    sel = (bits > t) | (tie & (idx <= J))
    out_ref[...] = sel.astype(jnp.float32)


def kernel(x, p, pool_mask):
    overlap = pl.pallas_call(
        _matvec_body,
        grid=(_NB, _NK),
        in_specs=[
            pl.BlockSpec((1, _BK), lambda j, k: (0, k)),
            pl.BlockSpec((_BK, _BN), lambda j, k: (k, j)),
            pl.BlockSpec((_BK, _BN), lambda j, k: (k, j)),
        ],
        out_specs=pl.BlockSpec((1, _BN), lambda j, k: (0, j)),
        out_shape=jax.ShapeDtypeStruct((1, _OUT), jnp.float32),
        compiler_params=pltpu.CompilerParams(vmem_limit_bytes=56 * 1024 * 1024),
    )(x, p, pool_mask)

    patch = pl.pallas_call(
        _patch_body,
        grid=(_NPATCH,),
        in_specs=[
            pl.BlockSpec((1, _IN), lambda q: (0, 0)),
            pl.BlockSpec((_IN, _BN), lambda q: (0, (13 * q) // 2)),
            pl.BlockSpec((_IN, _BN), lambda q: (0, (13 * q) // 2)),
        ],
        out_specs=pl.BlockSpec((1, _BN), lambda q: (0, q)),
        out_shape=jax.ShapeDtypeStruct((1, _NPATCH * _BN), jnp.float32),
        compiler_params=pltpu.CompilerParams(vmem_limit_bytes=56 * 1024 * 1024),
    )(x, p, pool_mask)

    activation = pl.pallas_call(
        _select_body,
        out_shape=jax.ShapeDtypeStruct((1, _OUT), jnp.float32),
    )(overlap, patch)
    return activation


# speedup vs baseline: 1.4356x; 1.4356x over previous
import jax
import jax.numpy as jnp
from jax.experimental import pallas as pl
from jax.experimental.pallas import tpu as pltpu

_IN = 4096
_OUT = 16384
_K = 328
_BN = 2048
_PBN = 512
_BK = 512
_NB = _OUT // _BN
_NK = _IN // _BK
_NTILE = 3328
_NPATCH = 5


def _matvec_body(x_ref, p_ref, m_ref, out_ref):
    k = pl.program_id(1)
    conn = jnp.where(p_ref[...] > 0.5, m_ref[...], 0.0)
    part = jnp.dot(x_ref[...], conn, preferred_element_type=jnp.float32)

    @pl.when(k == 0)
    def _():
        out_ref[...] = part

    @pl.when(k > 0)
    def _():
        out_ref[...] = out_ref[...] + part


def _patch_body(x_ref, p_ref, m_ref, out_ref):
    conn = jnp.where(p_ref[...] > 0.5, m_ref[...], 0.0)
    out_ref[...] = jnp.dot(x_ref[...], conn, preferred_element_type=jnp.float32)


def _select_body(ov_ref, patch_ref, out_ref):
    v = ov_ref[...]
    pieces = []
    for q in range(_NPATCH):
        off = (q % 2) * 256
        pieces.append(patch_ref[0:1, q * _PBN + off:q * _PBN + off + 256])
        pieces.append(v[:, q * _NTILE + 256:min((q + 1) * _NTILE, _OUT)])
    v = jnp.concatenate(pieces, axis=1)

    bits = jax.lax.bitcast_convert_type(v, jnp.int32)
    k = jnp.int32(_K)

    def val_step(_, carry):
        lo, hi = carry
        mid = lo + (hi - lo) // 2
        cnt = jnp.sum((bits >= mid).astype(jnp.int32))
        ge = cnt >= k
        return jnp.where(ge, mid, lo), jnp.where(ge, hi, mid)

    lo0 = jnp.int32(0)
    hi0 = jnp.int32(0x7F800000)
    t, _ = jax.lax.fori_loop(0, 31, val_step, (lo0, hi0))

    c_gt = jnp.sum((bits > t).astype(jnp.int32))
    need = k - c_gt

    idx = jax.lax.broadcasted_iota(jnp.int32, (1, _OUT), 1)
    tie = bits == t

    def idx_step(_, carry):
        lo, hi = carry
        mid = lo + (hi - lo) // 2
        cnt = jnp.sum((tie & (idx <= mid)).astype(jnp.int32))
        ge = cnt >= need
        return jnp.where(ge, lo, mid), jnp.where(ge, mid, hi)

    _, J = jax.lax.fori_loop(0, 14, idx_step, (jnp.int32(-1), jnp.int32(_OUT - 1)))

    sel = (bits > t) | (tie & (idx <= J))
    out_ref[...] = sel.astype(jnp.float32)


def kernel(x, p, pool_mask):
    overlap = pl.pallas_call(
        _matvec_body,
        grid=(_NB, _NK),
        in_specs=[
            pl.BlockSpec((1, _BK), lambda j, k: (0, k)),
            pl.BlockSpec((_BK, _BN), lambda j, k: (k, j)),
            pl.BlockSpec((_BK, _BN), lambda j, k: (k, j)),
        ],
        out_specs=pl.BlockSpec((1, _BN), lambda j, k: (0, j)),
        out_shape=jax.ShapeDtypeStruct((1, _OUT), jnp.float32),
        compiler_params=pltpu.CompilerParams(vmem_limit_bytes=56 * 1024 * 1024),
    )(x, p, pool_mask)

    patch = pl.pallas_call(
        _patch_body,
        grid=(_NPATCH,),
        in_specs=[
            pl.BlockSpec((1, _IN), lambda q: (0, 0)),
            pl.BlockSpec((_IN, _PBN), lambda q: (0, (13 * q) // 2)),
            pl.BlockSpec((_IN, _PBN), lambda q: (0, (13 * q) // 2)),
        ],
        out_specs=pl.BlockSpec((1, _PBN), lambda q: (0, q)),
        out_shape=jax.ShapeDtypeStruct((1, _NPATCH * _PBN), jnp.float32),
        compiler_params=pltpu.CompilerParams(vmem_limit_bytes=56 * 1024 * 1024),
    )(x, p, pool_mask)

    activation = pl.pallas_call(
        _select_body,
        out_shape=jax.ShapeDtypeStruct((1, _OUT), jnp.float32),
    )(overlap, patch)
    return activation
